# baseline (speedup 1.0000x reference)
import functools

import jax
import jax.numpy as jnp
from jax import lax
from jax.experimental import pallas as pl
from jax.experimental.pallas import tpu as pltpu
from jax.experimental.pallas import tpu_sc as plsc

_D = 768
_H = _D // 2
_C = 16
_NIDX = 6 * _C


def _layout_sc(bbox_cols, table_cols):
  n_tok = bbox_cols.shape[1]
  info = plsc.get_sparse_core_info()
  ns = info.num_subcores
  n_per = n_tok // ns
  n_chunks = n_per // _C
  mesh = plsc.VectorSubcoreMesh(core_axis_name="c", subcore_axis_name="s")

  @functools.partial(
      pl.kernel,
      mesh=mesh,
      out_type=jax.ShapeDtypeStruct((n_tok, _D), jnp.float32),
      scratch_types=[
          pltpu.VMEM((4, n_per), jnp.int32),
          pltpu.VMEM((_NIDX,), jnp.int32),
          pltpu.VMEM((_NIDX,), jnp.int32),
          pltpu.VMEM((_NIDX, _H), jnp.float32),
          pltpu.VMEM((_NIDX, _H), jnp.float32),
          pltpu.VMEM((_C, _H), jnp.float32),
          pltpu.SemaphoreType.DMA,
          pltpu.SemaphoreType.DMA,
      ],
  )
  def k(bbox_hbm, table_hbm, out_hbm, bbox_v, idx0, idx1, g0, g1, obuf,
        sem0, sem1):
    idxs = (idx0, idx1)
    gbufs = (g0, g1)
    sems = (sem0, sem1)
    cid = lax.axis_index("c")
    sid = lax.axis_index("s")
    tok_base = sid * n_per
    col_off = cid * _H
    row_off = cid * 4096
    pltpu.sync_copy(bbox_hbm.at[:, pl.ds(tok_base, n_per)], bbox_v)

    def start_gather(ci, p):
      off = ci * _C
      b0 = bbox_v[0, pl.ds(off, _C)]
      b1 = bbox_v[1, pl.ds(off, _C)]
      b2 = bbox_v[2, pl.ds(off, _C)]
      b3 = bbox_v[3, pl.ds(off, _C)]
      idxs[p][pl.ds(0, _C)] = b0 + row_off
      idxs[p][pl.ds(_C, _C)] = b2 + row_off
      idxs[p][pl.ds(2 * _C, _C)] = b1 + (1024 + row_off)
      idxs[p][pl.ds(3 * _C, _C)] = b3 + (1024 + row_off)
      idxs[p][pl.ds(4 * _C, _C)] = (b3 - b1) + (2048 + row_off)
      idxs[p][pl.ds(5 * _C, _C)] = (b2 - b0) + (3072 + row_off)
      pltpu.async_copy(table_hbm.at[idxs[p]], gbufs[p], sems[p])

    start_gather(0, 0)

    def pair(pi, carry):
      for h in range(2):
        ci = pi * 2 + h
        g = gbufs[h]
        pltpu.make_async_copy(table_hbm.at[idxs[h]], g, sems[h]).wait()
        nxt = ci + 1

        @pl.when(nxt < n_chunks)
        def _():
          start_gather(nxt, 1 - h)

        def tbody(t, c2):
          for m in range(_H // 16):
            sl = pl.ds(m * 16, 16)
            acc = g[t, sl] + g[_C + t, sl]
            acc = acc + g[2 * _C + t, sl]
            acc = acc + g[3 * _C + t, sl]
            acc = acc + g[4 * _C + t, sl]
            acc = acc + g[5 * _C + t, sl]
            obuf[t, sl] = acc
          return c2

        lax.fori_loop(0, _C, tbody, 0)
        pltpu.sync_copy(
            obuf, out_hbm.at[pl.ds(tok_base + ci * _C, _C),
                             pl.ds(col_off, _H)])
      return carry

    lax.fori_loop(0, n_chunks // 2, pair, 0)

  return k(bbox_cols, table_cols)


def _fuse_tc(x, layout, w_text, w_layout, bias, gamma, beta, out_buf,
             blk_off):
  n_tok = x.shape[0]
  tb = 512
  grid = (n_tok // tb,)

  def body(ob_ref, x_ref, l_ref, wt_ref, wl_ref, b_ref, g_ref, be_ref,
           o_ref):
    xv = x_ref[...]
    lv = l_ref[...]
    logits = (
        jnp.dot(xv, wt_ref[...], preferred_element_type=jnp.float32)
        + jnp.dot(lv, wl_ref[...], preferred_element_type=jnp.float32)
        + b_ref[...]
    )
    imp = jax.nn.sigmoid(logits)
    emb = xv * imp[:, 0:1] + lv * imp[:, 1:2]
    mean = jnp.mean(emb, axis=1, keepdims=True)
    cen = emb - mean
    var = jnp.mean(cen * cen, axis=1, keepdims=True)
    o_ref[...] = cen * lax.rsqrt(var + 1e-5) * g_ref[...] + be_ref[...]

  return pl.pallas_call(
      body,
      grid=grid,
      in_specs=[
          pl.BlockSpec(memory_space=pl.ANY),
          pl.BlockSpec((tb, _D), lambda i: (i, 0)),
          pl.BlockSpec((tb, _D), lambda i: (i, 0)),
          pl.BlockSpec((_D, 2), lambda i: (0, 0)),
          pl.BlockSpec((_D, 2), lambda i: (0, 0)),
          pl.BlockSpec((1, 2), lambda i: (0, 0)),
          pl.BlockSpec((1, _D), lambda i: (0, 0)),
          pl.BlockSpec((1, _D), lambda i: (0, 0)),
      ],
      out_specs=pl.BlockSpec((tb, _D), lambda i: (i + blk_off, 0)),
      out_shape=jax.ShapeDtypeStruct(out_buf.shape, jnp.float32),
      input_output_aliases={0: 0},
  )(out_buf, x, layout, w_text, w_layout, bias, gamma, beta)


def kernel(bbox, inputs_embeds, x_table, y_table, h_table, w_table,
           ln_gamma, ln_beta, lin_W, lin_b):
  b, s, d = inputs_embeds.shape
  n_tok = b * s
  bbox_cols = bbox.reshape(n_tok, 4).T.astype(jnp.int32)
  table_cols = jnp.concatenate(
      [x_table[:, :_H], y_table[:, :_H], h_table[:, :_H], w_table[:, :_H],
       x_table[:, _H:], y_table[:, _H:], h_table[:, _H:], w_table[:, _H:]],
      axis=0)
  x = inputs_embeds.reshape(n_tok, d)
  w_text = lin_W[:, :d].T
  w_layout = lin_W[:, d:].T
  bias = lin_b.reshape(1, 2)
  gamma = ln_gamma.reshape(1, d)
  beta = ln_beta.reshape(1, d)
  half = n_tok // 2
  layout0 = _layout_sc(bbox_cols[:, :half], table_cols)
  layout1 = _layout_sc(bbox_cols[:, half:], table_cols)
  out_buf = jnp.zeros((n_tok, d), jnp.float32)
  out_buf = _fuse_tc(x[:half], layout0, w_text, w_layout, bias, gamma,
                     beta, out_buf, 0)
  out_buf = _fuse_tc(x[half:], layout1, w_text, w_layout, bias, gamma,
                     beta, out_buf, half // 512)
  return out_buf.reshape(b, s, d)


# speedup vs baseline: 1.1020x; 1.1020x over previous
import functools

import jax
import jax.numpy as jnp
from jax import lax
from jax.experimental import pallas as pl
from jax.experimental.pallas import tpu as pltpu
from jax.experimental.pallas import tpu_sc as plsc

_D = 768
_H = _D // 2
_C = 16
_NIDX = 6 * _C


def _layout_sc(bbox_cols, table_cols):
  n_tok = bbox_cols.shape[1]
  info = plsc.get_sparse_core_info()
  ns = info.num_subcores
  n_per = n_tok // ns
  n_chunks = n_per // _C
  mesh = plsc.VectorSubcoreMesh(core_axis_name="c", subcore_axis_name="s")

  @functools.partial(
      pl.kernel,
      mesh=mesh,
      out_type=jax.ShapeDtypeStruct((n_tok, _D), jnp.float32),
      scratch_types=[
          pltpu.VMEM((4, n_per), jnp.int32),
          pltpu.VMEM((_NIDX,), jnp.int32),
          pltpu.VMEM((_NIDX,), jnp.int32),
          pltpu.VMEM((_NIDX, _H), jnp.float32),
          pltpu.VMEM((_NIDX, _H), jnp.float32),
          pltpu.VMEM((_C, _H), jnp.float32),
          pltpu.SemaphoreType.DMA,
          pltpu.SemaphoreType.DMA,
      ],
  )
  def k(bbox_hbm, table_hbm, out_hbm, bbox_v, idx0, idx1, g0, g1, obuf,
        sem0, sem1):
    idxs = (idx0, idx1)
    gbufs = (g0, g1)
    sems = (sem0, sem1)
    cid = lax.axis_index("c")
    sid = lax.axis_index("s")
    tok_base = sid * n_per
    col_off = cid * _H
    row_off = cid * 4096
    pltpu.sync_copy(bbox_hbm.at[:, pl.ds(tok_base, n_per)], bbox_v)

    def start_gather(ci, p):
      off = ci * _C
      b0 = bbox_v[0, pl.ds(off, _C)]
      b1 = bbox_v[1, pl.ds(off, _C)]
      b2 = bbox_v[2, pl.ds(off, _C)]
      b3 = bbox_v[3, pl.ds(off, _C)]
      idxs[p][pl.ds(0, _C)] = b0 + row_off
      idxs[p][pl.ds(_C, _C)] = b2 + row_off
      idxs[p][pl.ds(2 * _C, _C)] = b1 + (1024 + row_off)
      idxs[p][pl.ds(3 * _C, _C)] = b3 + (1024 + row_off)
      idxs[p][pl.ds(4 * _C, _C)] = (b3 - b1) + (2048 + row_off)
      idxs[p][pl.ds(5 * _C, _C)] = (b2 - b0) + (3072 + row_off)
      pltpu.async_copy(table_hbm.at[idxs[p]], gbufs[p], sems[p])

    start_gather(0, 0)

    def pair(pi, carry):
      for h in range(2):
        ci = pi * 2 + h
        g = gbufs[h]
        pltpu.make_async_copy(table_hbm.at[idxs[h]], g, sems[h]).wait()
        nxt = ci + 1

        @pl.when(nxt < n_chunks)
        def _():
          start_gather(nxt, 1 - h)

        def tbody(t, c2):
          for m in range(_H // 16):
            sl = pl.ds(m * 16, 16)
            acc = g[t, sl] + g[_C + t, sl]
            acc = acc + g[2 * _C + t, sl]
            acc = acc + g[3 * _C + t, sl]
            acc = acc + g[4 * _C + t, sl]
            acc = acc + g[5 * _C + t, sl]
            obuf[t, sl] = acc
          return c2

        lax.fori_loop(0, _C, tbody, 0)
        pltpu.sync_copy(
            obuf, out_hbm.at[pl.ds(tok_base + ci * _C, _C),
                             pl.ds(col_off, _H)])
      return carry

    lax.fori_loop(0, n_chunks // 2, pair, 0)

  return k(bbox_cols, table_cols)


def _fuse_tc(x, layout, w_text, w_layout, bias, gamma, beta):
  n_tok = x.shape[0]
  tb = 512
  grid = (n_tok // tb,)

  def body(x_ref, l_ref, wt_ref, wl_ref, b_ref, g_ref, be_ref, o_ref):
    xv = x_ref[...]
    lv = l_ref[...]
    logits = (
        jnp.dot(xv, wt_ref[...], preferred_element_type=jnp.float32)
        + jnp.dot(lv, wl_ref[...], preferred_element_type=jnp.float32)
        + b_ref[...]
    )
    imp = jax.nn.sigmoid(logits)
    emb = xv * imp[:, 0:1] + lv * imp[:, 1:2]
    mean = jnp.mean(emb, axis=1, keepdims=True)
    cen = emb - mean
    var = jnp.mean(cen * cen, axis=1, keepdims=True)
    o_ref[...] = cen * lax.rsqrt(var + 1e-5) * g_ref[...] + be_ref[...]

  return pl.pallas_call(
      body,
      grid=grid,
      in_specs=[
          pl.BlockSpec((tb, _D), lambda i: (i, 0)),
          pl.BlockSpec((tb, _D), lambda i: (i, 0)),
          pl.BlockSpec((_D, 2), lambda i: (0, 0)),
          pl.BlockSpec((_D, 2), lambda i: (0, 0)),
          pl.BlockSpec((1, 2), lambda i: (0, 0)),
          pl.BlockSpec((1, _D), lambda i: (0, 0)),
          pl.BlockSpec((1, _D), lambda i: (0, 0)),
      ],
      out_specs=pl.BlockSpec((tb, _D), lambda i: (i, 0)),
      out_shape=jax.ShapeDtypeStruct((n_tok, _D), jnp.float32),
  )(x, layout, w_text, w_layout, bias, gamma, beta)


def kernel(bbox, inputs_embeds, x_table, y_table, h_table, w_table,
           ln_gamma, ln_beta, lin_W, lin_b):
  b, s, d = inputs_embeds.shape
  n_tok = b * s
  bbox_cols = bbox.reshape(n_tok, 4).T.astype(jnp.int32)
  table_cols = jnp.concatenate(
      [x_table[:, :_H], y_table[:, :_H], h_table[:, :_H], w_table[:, :_H],
       x_table[:, _H:], y_table[:, _H:], h_table[:, _H:], w_table[:, _H:]],
      axis=0)
  layout = _layout_sc(bbox_cols, table_cols)
  out = _fuse_tc(
      inputs_embeds.reshape(n_tok, d),
      layout,
      lin_W[:, :d].T,
      lin_W[:, d:].T,
      lin_b.reshape(1, 2),
      ln_gamma.reshape(1, d),
      ln_beta.reshape(1, d),
  )
  return out.reshape(b, s, d)


# speedup vs baseline: 1.1387x; 1.0333x over previous
import functools

import jax
import jax.numpy as jnp
from jax import lax
from jax.experimental import pallas as pl
from jax.experimental.pallas import tpu as pltpu
from jax.experimental.pallas import tpu_sc as plsc

_D = 768
_H = _D // 2
_C = 16
_NIDX = 6 * _C


def _layout_sc(bbox_cols, table_cols):
  n_tok = bbox_cols.shape[1]
  info = plsc.get_sparse_core_info()
  ns = info.num_subcores
  n_per = n_tok // ns
  n_chunks = n_per // _C
  mesh = plsc.VectorSubcoreMesh(core_axis_name="c", subcore_axis_name="s")

  @functools.partial(
      pl.kernel,
      mesh=mesh,
      out_type=jax.ShapeDtypeStruct((n_tok, _D), jnp.float32),
      scratch_types=[
          pltpu.VMEM((4, n_per), jnp.int32),
          pltpu.VMEM((_NIDX,), jnp.int32),
          pltpu.VMEM((_NIDX,), jnp.int32),
          pltpu.VMEM((_NIDX, _H), jnp.float32),
          pltpu.VMEM((_NIDX, _H), jnp.float32),
          pltpu.VMEM((_C, _H), jnp.float32),
          pltpu.SemaphoreType.DMA,
          pltpu.SemaphoreType.DMA,
      ],
  )
  def k(bbox_hbm, table_hbm, out_hbm, bbox_v, idx0, idx1, g0, g1, obuf,
        sem0, sem1):
    idxs = (idx0, idx1)
    gbufs = (g0, g1)
    sems = (sem0, sem1)
    cid = lax.axis_index("c")
    sid = lax.axis_index("s")
    tok_base = sid * n_per
    col_off = cid * _H
    row_off = cid * 4096
    pltpu.sync_copy(bbox_hbm.at[:, pl.ds(tok_base, n_per)], bbox_v)

    def start_gather(ci, p):
      off = ci * _C
      b0 = bbox_v[0, pl.ds(off, _C)]
      b1 = bbox_v[1, pl.ds(off, _C)]
      b2 = bbox_v[2, pl.ds(off, _C)]
      b3 = bbox_v[3, pl.ds(off, _C)]
      idxs[p][pl.ds(0, _C)] = b0 + row_off
      idxs[p][pl.ds(_C, _C)] = b2 + row_off
      idxs[p][pl.ds(2 * _C, _C)] = b1 + (1024 + row_off)
      idxs[p][pl.ds(3 * _C, _C)] = b3 + (1024 + row_off)
      idxs[p][pl.ds(4 * _C, _C)] = (b3 - b1) + (2048 + row_off)
      idxs[p][pl.ds(5 * _C, _C)] = (b2 - b0) + (3072 + row_off)
      pltpu.async_copy(table_hbm.at[idxs[p]], gbufs[p], sems[p])

    start_gather(0, 0)

    def pair(pi, carry):
      for h in range(2):
        ci = pi * 2 + h
        g = gbufs[h]
        pltpu.make_async_copy(table_hbm.at[idxs[h]], g, sems[h]).wait()
        nxt = ci + 1

        @pl.when(nxt < n_chunks)
        def _():
          start_gather(nxt, 1 - h)

        def tbody(t, c2):
          for m in range(_H // 16):
            sl = pl.ds(m * 16, 16)
            acc = g[t, sl] + g[_C + t, sl]
            acc = acc + g[2 * _C + t, sl]
            acc = acc + g[3 * _C + t, sl]
            acc = acc + g[4 * _C + t, sl]
            acc = acc + g[5 * _C + t, sl]
            obuf[t, sl] = acc
          return c2

        lax.fori_loop(0, _C, tbody, 0)
        pltpu.sync_copy(
            obuf, out_hbm.at[pl.ds(tok_base + ci * _C, _C),
                             pl.ds(col_off, _H)])
      return carry

    lax.fori_loop(0, n_chunks // 2, pair, 0)

  return k(bbox_cols, table_cols)


def _fuse_tc(x, layout, w_text, w_layout, bias, gamma, beta):
  n_tok = x.shape[0]
  tb = 1024
  grid = (n_tok // tb,)

  def body(x_ref, l_ref, wt_ref, wl_ref, b_ref, g_ref, be_ref, o_ref):
    xv = x_ref[...]
    lv = l_ref[...]
    logits = (
        jnp.dot(xv, wt_ref[...], preferred_element_type=jnp.float32)
        + jnp.dot(lv, wl_ref[...], preferred_element_type=jnp.float32)
        + b_ref[...]
    )
    imp = jax.nn.sigmoid(logits)
    emb = xv * imp[:, 0:1] + lv * imp[:, 1:2]
    mean = jnp.mean(emb, axis=1, keepdims=True)
    cen = emb - mean
    var = jnp.mean(cen * cen, axis=1, keepdims=True)
    o_ref[...] = cen * lax.rsqrt(var + 1e-5) * g_ref[...] + be_ref[...]

  return pl.pallas_call(
      body,
      grid=grid,
      in_specs=[
          pl.BlockSpec((tb, _D), lambda i: (i, 0)),
          pl.BlockSpec((tb, _D), lambda i: (i, 0)),
          pl.BlockSpec((_D, 2), lambda i: (0, 0)),
          pl.BlockSpec((_D, 2), lambda i: (0, 0)),
          pl.BlockSpec((1, 2), lambda i: (0, 0)),
          pl.BlockSpec((1, _D), lambda i: (0, 0)),
          pl.BlockSpec((1, _D), lambda i: (0, 0)),
      ],
      out_specs=pl.BlockSpec((tb, _D), lambda i: (i, 0)),
      out_shape=jax.ShapeDtypeStruct((n_tok, _D), jnp.float32),
  )(x, layout, w_text, w_layout, bias, gamma, beta)


def kernel(bbox, inputs_embeds, x_table, y_table, h_table, w_table,
           ln_gamma, ln_beta, lin_W, lin_b):
  b, s, d = inputs_embeds.shape
  n_tok = b * s
  bbox_cols = bbox.reshape(n_tok, 4).T.astype(jnp.int32)
  table_cols = jnp.concatenate(
      [x_table[:, :_H], y_table[:, :_H], h_table[:, :_H], w_table[:, :_H],
       x_table[:, _H:], y_table[:, _H:], h_table[:, _H:], w_table[:, _H:]],
      axis=0)
  layout = _layout_sc(bbox_cols, table_cols)
  out = _fuse_tc(
      inputs_embeds.reshape(n_tok, d),
      layout,
      lin_W[:, :d].T,
      lin_W[:, d:].T,
      lin_b.reshape(1, 2),
      ln_gamma.reshape(1, d),
      ln_beta.reshape(1, d),
  )
  return out.reshape(b, s, d)


# speedup vs baseline: 1.1514x; 1.0112x over previous
import functools

import jax
import jax.numpy as jnp
from jax import lax
from jax.experimental import pallas as pl
from jax.experimental.pallas import tpu as pltpu
from jax.experimental.pallas import tpu_sc as plsc

_D = 768
_H = _D // 2
_C = 16
_NIDX = 6 * _C


def _layout_sc(bbox_cols, table_cols):
  n_tok = bbox_cols.shape[1]
  info = plsc.get_sparse_core_info()
  ns = info.num_subcores
  n_per = n_tok // ns
  n_chunks = n_per // _C
  mesh = plsc.VectorSubcoreMesh(core_axis_name="c", subcore_axis_name="s")

  @functools.partial(
      pl.kernel,
      mesh=mesh,
      out_type=jax.ShapeDtypeStruct((n_tok, _D), jnp.float32),
      scratch_types=[
          pltpu.VMEM((4, n_per), jnp.int32),
          pltpu.VMEM((_NIDX,), jnp.int32),
          pltpu.VMEM((_NIDX,), jnp.int32),
          pltpu.VMEM((_NIDX, _H), jnp.float32),
          pltpu.VMEM((_NIDX, _H), jnp.float32),
          pltpu.VMEM((_C, _H), jnp.float32),
          pltpu.SemaphoreType.DMA,
          pltpu.SemaphoreType.DMA,
      ],
  )
  def k(bbox_hbm, table_hbm, out_hbm, bbox_v, idx0, idx1, g0, g1, obuf,
        sem0, sem1):
    idxs = (idx0, idx1)
    gbufs = (g0, g1)
    sems = (sem0, sem1)
    cid = lax.axis_index("c")
    sid = lax.axis_index("s")
    tok_base = sid * n_per
    col_off = cid * _H
    row_off = cid * 4096
    pltpu.sync_copy(bbox_hbm.at[:, pl.ds(tok_base, n_per)], bbox_v)

    def start_gather(ci, p):
      off = ci * _C
      b0 = bbox_v[0, pl.ds(off, _C)]
      b1 = bbox_v[1, pl.ds(off, _C)]
      b2 = bbox_v[2, pl.ds(off, _C)]
      b3 = bbox_v[3, pl.ds(off, _C)]
      idxs[p][pl.ds(0, _C)] = b0 + row_off
      idxs[p][pl.ds(_C, _C)] = b2 + row_off
      idxs[p][pl.ds(2 * _C, _C)] = b1 + (1024 + row_off)
      idxs[p][pl.ds(3 * _C, _C)] = b3 + (1024 + row_off)
      idxs[p][pl.ds(4 * _C, _C)] = (b3 - b1) + (2048 + row_off)
      idxs[p][pl.ds(5 * _C, _C)] = (b2 - b0) + (3072 + row_off)
      pltpu.async_copy(table_hbm.at[idxs[p]], gbufs[p], sems[p])

    start_gather(0, 0)

    def pair(pi, carry):
      for h in range(2):
        ci = pi * 2 + h
        g = gbufs[h]
        pltpu.make_async_copy(table_hbm.at[idxs[h]], g, sems[h]).wait()
        nxt = ci + 1

        @pl.when(nxt < n_chunks)
        def _():
          start_gather(nxt, 1 - h)

        def tbody(t, c2):
          for m in range(_H // 16):
            sl = pl.ds(m * 16, 16)
            acc = g[t, sl] + g[_C + t, sl]
            acc = acc + g[2 * _C + t, sl]
            acc = acc + g[3 * _C + t, sl]
            acc = acc + g[4 * _C + t, sl]
            acc = acc + g[5 * _C + t, sl]
            obuf[t, sl] = acc
          return c2

        lax.fori_loop(0, _C, tbody, 0)
        pltpu.sync_copy(
            obuf, out_hbm.at[pl.ds(tok_base + ci * _C, _C),
                             pl.ds(col_off, _H)])
      return carry

    lax.fori_loop(0, n_chunks // 2, pair, 0)

  return k(bbox_cols, table_cols)


def _fuse_tc(x, layout, w_text, w_layout, bias, gamma, beta):
  n_tok = x.shape[0]
  tb = 2048
  grid = (n_tok // tb,)

  def body(x_ref, l_ref, wt_ref, wl_ref, b_ref, g_ref, be_ref, o_ref):
    xv = x_ref[...]
    lv = l_ref[...]
    logits = (
        jnp.dot(xv, wt_ref[...], preferred_element_type=jnp.float32)
        + jnp.dot(lv, wl_ref[...], preferred_element_type=jnp.float32)
        + b_ref[...]
    )
    imp = jax.nn.sigmoid(logits)
    emb = xv * imp[:, 0:1] + lv * imp[:, 1:2]
    mean = jnp.mean(emb, axis=1, keepdims=True)
    cen = emb - mean
    var = jnp.mean(cen * cen, axis=1, keepdims=True)
    o_ref[...] = cen * lax.rsqrt(var + 1e-5) * g_ref[...] + be_ref[...]

  return pl.pallas_call(
      body,
      grid=grid,
      in_specs=[
          pl.BlockSpec((tb, _D), lambda i: (i, 0)),
          pl.BlockSpec((tb, _D), lambda i: (i, 0)),
          pl.BlockSpec((_D, 2), lambda i: (0, 0)),
          pl.BlockSpec((_D, 2), lambda i: (0, 0)),
          pl.BlockSpec((1, 2), lambda i: (0, 0)),
          pl.BlockSpec((1, _D), lambda i: (0, 0)),
          pl.BlockSpec((1, _D), lambda i: (0, 0)),
      ],
      out_specs=pl.BlockSpec((tb, _D), lambda i: (i, 0)),
      out_shape=jax.ShapeDtypeStruct((n_tok, _D), jnp.float32),
  )(x, layout, w_text, w_layout, bias, gamma, beta)


def kernel(bbox, inputs_embeds, x_table, y_table, h_table, w_table,
           ln_gamma, ln_beta, lin_W, lin_b):
  b, s, d = inputs_embeds.shape
  n_tok = b * s
  bbox_cols = bbox.reshape(n_tok, 4).T.astype(jnp.int32)
  table_cols = jnp.concatenate(
      [x_table[:, :_H], y_table[:, :_H], h_table[:, :_H], w_table[:, :_H],
       x_table[:, _H:], y_table[:, _H:], h_table[:, _H:], w_table[:, _H:]],
      axis=0)
  layout = _layout_sc(bbox_cols, table_cols)
  out = _fuse_tc(
      inputs_embeds.reshape(n_tok, d),
      layout,
      lin_W[:, :d].T,
      lin_W[:, d:].T,
      lin_b.reshape(1, 2),
      ln_gamma.reshape(1, d),
      ln_beta.reshape(1, d),
  )
  return out.reshape(b, s, d)


# speedup vs baseline: 1.1578x; 1.0055x over previous
import functools

import jax
import jax.numpy as jnp
from jax import lax
from jax.experimental import pallas as pl
from jax.experimental.pallas import tpu as pltpu
from jax.experimental.pallas import tpu_sc as plsc

_D = 768
_H = _D // 2
_C = 16
_NIDX = 6 * _C


def _layout_sc(bbox_cols, table_cols):
  n_tok = bbox_cols.shape[1]
  info = plsc.get_sparse_core_info()
  ns = info.num_subcores
  n_per = n_tok // ns
  n_chunks = n_per // _C
  mesh = plsc.VectorSubcoreMesh(core_axis_name="c", subcore_axis_name="s")

  @functools.partial(
      pl.kernel,
      mesh=mesh,
      out_type=jax.ShapeDtypeStruct((n_tok, _D), jnp.float32),
      scratch_types=[
          pltpu.VMEM((4, n_per), jnp.int32),
          pltpu.VMEM((_NIDX,), jnp.int32),
          pltpu.VMEM((_NIDX,), jnp.int32),
          pltpu.VMEM((_NIDX, _H), jnp.float32),
          pltpu.VMEM((_NIDX, _H), jnp.float32),
          pltpu.VMEM((_C, _H), jnp.float32),
          pltpu.VMEM((_C, _H), jnp.float32),
          pltpu.SemaphoreType.DMA,
          pltpu.SemaphoreType.DMA,
          pltpu.SemaphoreType.DMA,
          pltpu.SemaphoreType.DMA,
      ],
  )
  def k(bbox_hbm, table_hbm, out_hbm, bbox_v, idx0, idx1, g0, g1, obuf0,
        obuf1, sem0, sem1, sem2, sem3):
    idxs = (idx0, idx1)
    gbufs = (g0, g1)
    obufs = (obuf0, obuf1)
    sems = (sem0, sem1)
    fsems = (sem2, sem3)
    cid = lax.axis_index("c")
    sid = lax.axis_index("s")
    tok_base = sid * n_per
    col_off = cid * _H
    row_off = cid * 4096
    pltpu.sync_copy(bbox_hbm.at[:, pl.ds(tok_base, n_per)], bbox_v)

    def start_gather(ci, p):
      off = ci * _C
      b0 = bbox_v[0, pl.ds(off, _C)]
      b1 = bbox_v[1, pl.ds(off, _C)]
      b2 = bbox_v[2, pl.ds(off, _C)]
      b3 = bbox_v[3, pl.ds(off, _C)]
      idxs[p][pl.ds(0, _C)] = b0 + row_off
      idxs[p][pl.ds(_C, _C)] = b2 + row_off
      idxs[p][pl.ds(2 * _C, _C)] = b1 + (1024 + row_off)
      idxs[p][pl.ds(3 * _C, _C)] = b3 + (1024 + row_off)
      idxs[p][pl.ds(4 * _C, _C)] = (b3 - b1) + (2048 + row_off)
      idxs[p][pl.ds(5 * _C, _C)] = (b2 - b0) + (3072 + row_off)
      pltpu.async_copy(table_hbm.at[idxs[p]], gbufs[p], sems[p])

    start_gather(0, 0)

    def pair(pi, carry):
      for h in range(2):
        ci = pi * 2 + h
        g = gbufs[h]
        pltpu.make_async_copy(table_hbm.at[idxs[h]], g, sems[h]).wait()
        nxt = ci + 1

        @pl.when(nxt < n_chunks)
        def _():
          start_gather(nxt, 1 - h)

        @pl.when(ci >= 2)
        def _():
          pltpu.make_async_copy(
              obufs[h], out_hbm.at[pl.ds(tok_base, _C), pl.ds(col_off, _H)],
              fsems[h]).wait()

        def tbody(t, c2):
          for m in range(_H // 16):
            sl = pl.ds(m * 16, 16)
            acc = g[t, sl] + g[_C + t, sl]
            acc = acc + g[2 * _C + t, sl]
            acc = acc + g[3 * _C + t, sl]
            acc = acc + g[4 * _C + t, sl]
            acc = acc + g[5 * _C + t, sl]
            obufs[h][t, sl] = acc
          return c2

        lax.fori_loop(0, _C, tbody, 0)
        pltpu.async_copy(
            obufs[h], out_hbm.at[pl.ds(tok_base + ci * _C, _C),
                                 pl.ds(col_off, _H)], fsems[h])
      return carry

    lax.fori_loop(0, n_chunks // 2, pair, 0)
    for h in range(2):
      pltpu.make_async_copy(
          obufs[h], out_hbm.at[pl.ds(tok_base, _C), pl.ds(col_off, _H)],
          fsems[h]).wait()

  return k(bbox_cols, table_cols)


def _fuse_tc(x, layout, w_text, w_layout, bias, gamma, beta):
  n_tok = x.shape[0]
  tb = 2048
  grid = (n_tok // tb,)

  def body(x_ref, l_ref, wt_ref, wl_ref, b_ref, g_ref, be_ref, o_ref):
    xv = x_ref[...]
    lv = l_ref[...]
    logits = (
        jnp.dot(xv, wt_ref[...], preferred_element_type=jnp.float32)
        + jnp.dot(lv, wl_ref[...], preferred_element_type=jnp.float32)
        + b_ref[...]
    )
    imp = jax.nn.sigmoid(logits)
    emb = xv * imp[:, 0:1] + lv * imp[:, 1:2]
    mean = jnp.mean(emb, axis=1, keepdims=True)
    cen = emb - mean
    var = jnp.mean(cen * cen, axis=1, keepdims=True)
    o_ref[...] = cen * lax.rsqrt(var + 1e-5) * g_ref[...] + be_ref[...]

  return pl.pallas_call(
      body,
      grid=grid,
      in_specs=[
          pl.BlockSpec((tb, _D), lambda i: (i, 0)),
          pl.BlockSpec((tb, _D), lambda i: (i, 0)),
          pl.BlockSpec((_D, 2), lambda i: (0, 0)),
          pl.BlockSpec((_D, 2), lambda i: (0, 0)),
          pl.BlockSpec((1, 2), lambda i: (0, 0)),
          pl.BlockSpec((1, _D), lambda i: (0, 0)),
          pl.BlockSpec((1, _D), lambda i: (0, 0)),
      ],
      out_specs=pl.BlockSpec((tb, _D), lambda i: (i, 0)),
      out_shape=jax.ShapeDtypeStruct((n_tok, _D), jnp.float32),
  )(x, layout, w_text, w_layout, bias, gamma, beta)


def kernel(bbox, inputs_embeds, x_table, y_table, h_table, w_table,
           ln_gamma, ln_beta, lin_W, lin_b):
  b, s, d = inputs_embeds.shape
  n_tok = b * s
  bbox_cols = bbox.reshape(n_tok, 4).T.astype(jnp.int32)
  table_cols = jnp.concatenate(
      [x_table[:, :_H], y_table[:, :_H], h_table[:, :_H], w_table[:, :_H],
       x_table[:, _H:], y_table[:, _H:], h_table[:, _H:], w_table[:, _H:]],
      axis=0)
  layout = _layout_sc(bbox_cols, table_cols)
  out = _fuse_tc(
      inputs_embeds.reshape(n_tok, d),
      layout,
      lin_W[:, :d].T,
      lin_W[:, d:].T,
      lin_b.reshape(1, 2),
      ln_gamma.reshape(1, d),
      ln_beta.reshape(1, d),
  )
  return out.reshape(b, s, d)
